# baseline (speedup 1.0000x reference)
import functools

import jax
import jax.numpy as jnp
from jax.experimental import pallas as pl

B, S, H, I, E, TOP_K = 1, 2048, 768, 2048, 8, 2
T = B * S
TM = 256


def _router_body(x_ref, wr_ref, comb_ref):
    x = x_ref[...]
    logits = jnp.dot(x, wr_ref[...], preferred_element_type=jnp.float32)
    lmax = jnp.max(logits, axis=1, keepdims=True)
    z = jnp.exp(logits - lmax)
    iota = jax.lax.broadcasted_iota(jnp.int32, (TM, E), 1)
    e1 = jnp.argmax(z, axis=1)[:, None]
    m1 = jnp.max(z, axis=1, keepdims=True)
    z2 = jnp.where(iota == e1, -jnp.inf, z)
    e2 = jnp.argmax(z2, axis=1)[:, None]
    m2 = jnp.max(z2, axis=1, keepdims=True)
    sel = (iota == e1) | (iota == e2)
    comb_ref[...] = jnp.where(sel, z / (m1 + m2), 0.0)


def _moe_body(xb_ref, comb_ref, wg_ref, wu_ref, wd_ref, out_ref):
    e = pl.program_id(1)
    xb = xb_ref[...]
    gate = jnp.dot(xb, wg_ref[...], preferred_element_type=jnp.float32)
    up = jnp.dot(xb, wu_ref[...], preferred_element_type=jnp.float32)
    inter = up * gate * jax.nn.sigmoid(gate)
    y = jnp.dot(inter, wd_ref[...], preferred_element_type=jnp.float32)
    iota = jax.lax.broadcasted_iota(jnp.int32, (TM, E), 1)
    c = jnp.sum(jnp.where(iota == e, comb_ref[...], 0.0), axis=1, keepdims=True)
    contrib = c * y

    @pl.when(e == 0)
    def _():
        out_ref[...] = contrib

    @pl.when(e != 0)
    def _():
        out_ref[...] += contrib


def kernel(hidden_states, Wr, Wg, Wu, Wd):
    b, s, h = hidden_states.shape
    x = hidden_states.reshape(T, H)

    combine = pl.pallas_call(
        _router_body,
        grid=(T // TM,),
        in_specs=[
            pl.BlockSpec((TM, H), lambda i: (i, 0)),
            pl.BlockSpec((H, E), lambda i: (0, 0)),
        ],
        out_specs=pl.BlockSpec((TM, E), lambda i: (i, 0)),
        out_shape=jax.ShapeDtypeStruct((T, E), jnp.float32),
    )(x, Wr)

    out = pl.pallas_call(
        _moe_body,
        grid=(T // TM, E),
        in_specs=[
            pl.BlockSpec((TM, H), lambda i, e: (i, 0)),
            pl.BlockSpec((TM, E), lambda i, e: (i, 0)),
            pl.BlockSpec((None, H, I), lambda i, e: (e, 0, 0)),
            pl.BlockSpec((None, H, I), lambda i, e: (e, 0, 0)),
            pl.BlockSpec((None, I, H), lambda i, e: (e, 0, 0)),
        ],
        out_specs=pl.BlockSpec((TM, H), lambda i, e: (i, 0)),
        out_shape=jax.ShapeDtypeStruct((T, H), jnp.float32),
    )(x, combine, Wg, Wu, Wd)

    return out.reshape(b, s, h)


# speedup vs baseline: 1.2473x; 1.2473x over previous
import functools

import jax
import jax.numpy as jnp
from jax.experimental import pallas as pl
from jax.experimental.pallas import tpu as pltpu

B, S, H, I, E, TOP_K = 1, 2048, 768, 2048, 8, 2
T = B * S
P = T * TOP_K
TM = 256
TM2 = 128
NT = P // TM2 + E
PPAD = NT * TM2


def _router_body(x_ref, wr_ref, ei_ref, ew_ref):
    x = x_ref[...]
    logits = jnp.dot(x, wr_ref[...], preferred_element_type=jnp.float32)
    lmax = jnp.max(logits, axis=1, keepdims=True)
    z = jnp.exp(logits - lmax)
    iota = jax.lax.broadcasted_iota(jnp.int32, (TM, E), 1)
    e1 = jnp.argmax(z, axis=1)[:, None]
    m1 = jnp.max(z, axis=1, keepdims=True)
    z2 = jnp.where(iota == e1, -jnp.inf, z)
    e2 = jnp.argmax(z2, axis=1)[:, None]
    m2 = jnp.max(z2, axis=1, keepdims=True)
    denom = m1 + m2
    ei_ref[...] = jnp.concatenate(
        [e1, e2, jnp.zeros((TM, E - 2), jnp.int32)], axis=1)
    ew_ref[...] = jnp.concatenate(
        [m1 / denom, m2 / denom, jnp.zeros((TM, E - 2), jnp.float32)], axis=1)


def _expert_of(i, off_ref):
    acc = 0
    for e in range(E - 1):
        acc += (i * TM2 >= off_ref[e]).astype(jnp.int32)
    return acc


def _gemm_body(off_ref, xp_ref, sw_ref, wg_ref, wu_ref, wd_ref, y_ref):
    i = pl.program_id(0)

    @pl.when(i * TM2 < off_ref[E - 1])
    def _():
        xt = xp_ref[...].astype(jnp.bfloat16)
        gate = jnp.dot(xt, wg_ref[...], preferred_element_type=jnp.float32)
        up = jnp.dot(xt, wu_ref[...], preferred_element_type=jnp.float32)
        inter = up * gate * jax.nn.sigmoid(gate)
        y = jnp.dot(inter.astype(jnp.bfloat16), wd_ref[...],
                    preferred_element_type=jnp.float32)
        y_ref[...] = sw_ref[...] * y


def kernel(hidden_states, Wr, Wg, Wu, Wd):
    b, s, h = hidden_states.shape
    x = hidden_states.reshape(T, H)

    ei, ew = pl.pallas_call(
        _router_body,
        grid=(T // TM,),
        in_specs=[
            pl.BlockSpec((TM, H), lambda i: (i, 0)),
            pl.BlockSpec((H, E), lambda i: (0, 0)),
        ],
        out_specs=[
            pl.BlockSpec((TM, E), lambda i: (i, 0)),
            pl.BlockSpec((TM, E), lambda i: (i, 0)),
        ],
        out_shape=[
            jax.ShapeDtypeStruct((T, E), jnp.int32),
            jax.ShapeDtypeStruct((T, E), jnp.float32),
        ],
    )(x, Wr)

    pe = jnp.concatenate([ei[:, 0], ei[:, 1]])
    pw = jnp.concatenate([ew[:, 0], ew[:, 1]])
    tok = jnp.concatenate([jnp.arange(T), jnp.arange(T)])
    onehot = (pe[:, None] == jnp.arange(E)[None, :]).astype(jnp.int32)
    ccum = jnp.cumsum(onehot, axis=0)
    cnt = ccum[-1]
    padded = ((cnt + TM2 - 1) // TM2) * TM2
    incl = jnp.cumsum(padded)
    excl = incl - padded
    rank = jnp.sum(onehot * ccum, axis=1) - 1
    slot = excl[pe] + rank
    off = incl.astype(jnp.int32)
    xp = jnp.zeros((PPAD, H), jnp.float32).at[slot].set(x[tok])
    sw = jnp.zeros((PPAD, 1), jnp.float32).at[slot, 0].set(pw)

    Wg_b = Wg.astype(jnp.bfloat16)
    Wu_b = Wu.astype(jnp.bfloat16)
    Wd_b = Wd.astype(jnp.bfloat16)

    yw = pl.pallas_call(
        _gemm_body,
        grid_spec=pltpu.PrefetchScalarGridSpec(
            num_scalar_prefetch=1,
            grid=(NT,),
            in_specs=[
                pl.BlockSpec((TM2, H), lambda i, off: (i, 0)),
                pl.BlockSpec((TM2, 1), lambda i, off: (i, 0)),
                pl.BlockSpec((None, H, I), lambda i, off: (_expert_of(i, off), 0, 0)),
                pl.BlockSpec((None, H, I), lambda i, off: (_expert_of(i, off), 0, 0)),
                pl.BlockSpec((None, I, H), lambda i, off: (_expert_of(i, off), 0, 0)),
            ],
            out_specs=pl.BlockSpec((TM2, H), lambda i, off: (i, 0)),
        ),
        out_shape=jax.ShapeDtypeStruct((PPAD, H), jnp.float32),
    )(off, xp, sw, Wg_b, Wu_b, Wd_b)

    out = yw[slot[:T]] + yw[slot[T:]]
    return out.reshape(b, s, h)


# speedup vs baseline: 1.4941x; 1.1979x over previous
import functools

import jax
import jax.numpy as jnp
from jax.experimental import pallas as pl
from jax.experimental.pallas import tpu as pltpu

B, S, H, I, E, TOP_K = 1, 2048, 768, 2048, 8, 2
T = B * S
P = T * TOP_K
TM = 256
TM2 = 128
NT = P // TM2 + E
PPAD = NT * TM2


def _router_body(x_ref, wr_ref, ei_ref, ew_ref):
    x = x_ref[...]
    logits = jnp.dot(x, wr_ref[...], preferred_element_type=jnp.float32)
    lmax = jnp.max(logits, axis=1, keepdims=True)
    z = jnp.exp(logits - lmax)
    iota = jax.lax.broadcasted_iota(jnp.int32, (TM, E), 1)
    e1 = jnp.argmax(z, axis=1)[:, None]
    m1 = jnp.max(z, axis=1, keepdims=True)
    z2 = jnp.where(iota == e1, -jnp.inf, z)
    e2 = jnp.argmax(z2, axis=1)[:, None]
    m2 = jnp.max(z2, axis=1, keepdims=True)
    denom = m1 + m2
    ei_ref[...] = jnp.concatenate(
        [e1, e2, jnp.zeros((TM, E - 2), jnp.int32)], axis=1)
    ew_ref[...] = jnp.concatenate(
        [m1 / denom, m2 / denom, jnp.zeros((TM, E - 2), jnp.float32)], axis=1)


def _expert_of(i, off_ref):
    acc = 0
    for e in range(E - 1):
        acc += (i * TM2 >= off_ref[e]).astype(jnp.int32)
    return acc


def _gemm_body(off_ref, xp_ref, sw_ref, wg_ref, wu_ref, wd_ref, y_ref):
    i = pl.program_id(0)

    @pl.when(i * TM2 < off_ref[E - 1])
    def _():
        xt = xp_ref[...]
        gate = jnp.dot(xt, wg_ref[...], preferred_element_type=jnp.float32)
        up = jnp.dot(xt, wu_ref[...], preferred_element_type=jnp.float32)
        inter = up * gate * jax.nn.sigmoid(gate)
        y = jnp.dot(inter, wd_ref[...],
                    preferred_element_type=jnp.float32)
        y_ref[...] = sw_ref[...] * y


def kernel(hidden_states, Wr, Wg, Wu, Wd):
    b, s, h = hidden_states.shape
    x = hidden_states.reshape(T, H)

    ei, ew = pl.pallas_call(
        _router_body,
        grid=(T // TM,),
        in_specs=[
            pl.BlockSpec((TM, H), lambda i: (i, 0)),
            pl.BlockSpec((H, E), lambda i: (0, 0)),
        ],
        out_specs=[
            pl.BlockSpec((TM, E), lambda i: (i, 0)),
            pl.BlockSpec((TM, E), lambda i: (i, 0)),
        ],
        out_shape=[
            jax.ShapeDtypeStruct((T, E), jnp.int32),
            jax.ShapeDtypeStruct((T, E), jnp.float32),
        ],
    )(x, Wr)

    pe = jnp.concatenate([ei[:, 0], ei[:, 1]])
    pw = jnp.concatenate([ew[:, 0], ew[:, 1]])
    tok = jnp.concatenate([jnp.arange(T), jnp.arange(T)])
    onehot = (pe[:, None] == jnp.arange(E)[None, :]).astype(jnp.int32)
    ccum = jnp.cumsum(onehot, axis=0)
    cnt = ccum[-1]
    padded = ((cnt + TM2 - 1) // TM2) * TM2
    incl = jnp.cumsum(padded)
    excl = incl - padded
    rank = jnp.sum(onehot * ccum, axis=1) - 1
    slot = excl[pe] + rank
    off = incl.astype(jnp.int32)
    xp = jnp.zeros((PPAD, H), jnp.float32).at[slot].set(x[tok])
    sw = jnp.zeros((PPAD, 1), jnp.float32).at[slot, 0].set(pw)

    yw = pl.pallas_call(
        _gemm_body,
        grid_spec=pltpu.PrefetchScalarGridSpec(
            num_scalar_prefetch=1,
            grid=(NT,),
            in_specs=[
                pl.BlockSpec((TM2, H), lambda i, off: (i, 0)),
                pl.BlockSpec((TM2, 1), lambda i, off: (i, 0)),
                pl.BlockSpec((None, H, I), lambda i, off: (_expert_of(i, off), 0, 0)),
                pl.BlockSpec((None, H, I), lambda i, off: (_expert_of(i, off), 0, 0)),
                pl.BlockSpec((None, I, H), lambda i, off: (_expert_of(i, off), 0, 0)),
            ],
            out_specs=pl.BlockSpec((TM2, H), lambda i, off: (i, 0)),
        ),
        out_shape=jax.ShapeDtypeStruct((PPAD, H), jnp.float32),
    )(off, xp, sw, Wg, Wu, Wd)

    out = yw[slot[:T]] + yw[slot[T:]]
    return out.reshape(b, s, h)


# speedup vs baseline: 1.5594x; 1.0437x over previous
import functools

import jax
import jax.numpy as jnp
from jax import lax
from jax.experimental import pallas as pl
from jax.experimental.pallas import tpu as pltpu
from jax.experimental.pallas import tpu_sc as plsc

B, S, H, I, E, TOP_K = 1, 2048, 768, 2048, 8, 2
T = B * S
P = T * TOP_K
TM = 256
TM2 = 128
NT = P // TM2 + E
PPAD = NT * TM2


def _router_body(x_ref, wr_ref, ei_ref, ew_ref):
    x = x_ref[...]
    logits = jnp.dot(x, wr_ref[...], preferred_element_type=jnp.float32)
    lmax = jnp.max(logits, axis=1, keepdims=True)
    z = jnp.exp(logits - lmax)
    iota = jax.lax.broadcasted_iota(jnp.int32, (TM, E), 1)
    e1 = jnp.argmax(z, axis=1)[:, None]
    m1 = jnp.max(z, axis=1, keepdims=True)
    z2 = jnp.where(iota == e1, -jnp.inf, z)
    e2 = jnp.argmax(z2, axis=1)[:, None]
    m2 = jnp.max(z2, axis=1, keepdims=True)
    denom = m1 + m2
    ei_ref[...] = jnp.concatenate(
        [e1, e2, jnp.zeros((TM, E - 2), jnp.int32)], axis=1)
    ew_ref[...] = jnp.concatenate(
        [m1 / denom, m2 / denom, jnp.zeros((TM, E - 2), jnp.float32)], axis=1)


def _expert_of(i, off_ref):
    acc = 0
    for e in range(E - 1):
        acc += (i * TM2 >= off_ref[e]).astype(jnp.int32)
    return acc


def _gemm_body(off_ref, xp_ref, sw_ref, wg_ref, wu_ref, wd_ref, y_ref):
    i = pl.program_id(0)

    @pl.when(i * TM2 < off_ref[E - 1])
    def _():
        xt = xp_ref[...]
        gate = jnp.dot(xt, wg_ref[...], preferred_element_type=jnp.float32)
        up = jnp.dot(xt, wu_ref[...], preferred_element_type=jnp.float32)
        inter = up * gate * jax.nn.sigmoid(gate)
        y = jnp.dot(inter, wd_ref[...],
                    preferred_element_type=jnp.float32)
        y_ref[...] = sw_ref[...] * y


TPW = T // 32


@functools.partial(
    pl.kernel,
    out_type=jax.ShapeDtypeStruct((T, H), jnp.float32),
    mesh=plsc.VectorSubcoreMesh(core_axis_name="c", subcore_axis_name="s"),
    scratch_types=[
        pltpu.VMEM((TPW,), jnp.int32),
        pltpu.VMEM((TPW,), jnp.int32),
        pltpu.VMEM((TPW, H), jnp.float32),
        pltpu.VMEM((TPW, H), jnp.float32),
        pltpu.SemaphoreType.DMA,
        pltpu.SemaphoreType.DMA,
    ],
)
def _combine_sc(y_hbm, slot_hbm, out_hbm, s0, s1, buf0, buf1, sem0, sem1):
    wid = lax.axis_index("s") * 2 + lax.axis_index("c")
    base = wid * TPW
    pltpu.sync_copy(slot_hbm.at[pl.ds(base, TPW)], s0)
    pltpu.sync_copy(slot_hbm.at[pl.ds(T + base, TPW)], s1)
    c0 = pltpu.async_copy(y_hbm.at[s0], buf0, sem0)
    c1 = pltpu.async_copy(y_hbm.at[s1], buf1, sem1)
    c0.wait()
    c1.wait()

    def body(j, carry):
        for c in range(H // 16):
            a = buf0[j, pl.ds(c * 16, 16)]
            b = buf1[j, pl.ds(c * 16, 16)]
            buf0[j, pl.ds(c * 16, 16)] = a + b
        return carry

    lax.fori_loop(0, TPW, body, 0)
    pltpu.sync_copy(buf0, out_hbm.at[pl.ds(base, TPW)])


def kernel(hidden_states, Wr, Wg, Wu, Wd):
    b, s, h = hidden_states.shape
    x = hidden_states.reshape(T, H)

    ei, ew = pl.pallas_call(
        _router_body,
        grid=(T // TM,),
        in_specs=[
            pl.BlockSpec((TM, H), lambda i: (i, 0)),
            pl.BlockSpec((H, E), lambda i: (0, 0)),
        ],
        out_specs=[
            pl.BlockSpec((TM, E), lambda i: (i, 0)),
            pl.BlockSpec((TM, E), lambda i: (i, 0)),
        ],
        out_shape=[
            jax.ShapeDtypeStruct((T, E), jnp.int32),
            jax.ShapeDtypeStruct((T, E), jnp.float32),
        ],
    )(x, Wr)

    pe = jnp.concatenate([ei[:, 0], ei[:, 1]])
    pw = jnp.concatenate([ew[:, 0], ew[:, 1]])
    tok = jnp.concatenate([jnp.arange(T), jnp.arange(T)])
    onehot = (pe[:, None] == jnp.arange(E)[None, :]).astype(jnp.int32)
    ccum = jnp.cumsum(onehot, axis=0)
    cnt = ccum[-1]
    padded = ((cnt + TM2 - 1) // TM2) * TM2
    incl = jnp.cumsum(padded)
    excl = incl - padded
    rank = jnp.sum(onehot * ccum, axis=1) - 1
    slot = excl[pe] + rank
    off = incl.astype(jnp.int32)
    xp = jnp.zeros((PPAD, H), jnp.float32).at[slot].set(x[tok])
    sw = jnp.zeros((PPAD, 1), jnp.float32).at[slot, 0].set(pw)

    yw = pl.pallas_call(
        _gemm_body,
        grid_spec=pltpu.PrefetchScalarGridSpec(
            num_scalar_prefetch=1,
            grid=(NT,),
            in_specs=[
                pl.BlockSpec((TM2, H), lambda i, off: (i, 0)),
                pl.BlockSpec((TM2, 1), lambda i, off: (i, 0)),
                pl.BlockSpec((None, H, I), lambda i, off: (_expert_of(i, off), 0, 0)),
                pl.BlockSpec((None, H, I), lambda i, off: (_expert_of(i, off), 0, 0)),
                pl.BlockSpec((None, I, H), lambda i, off: (_expert_of(i, off), 0, 0)),
            ],
            out_specs=pl.BlockSpec((TM2, H), lambda i, off: (i, 0)),
        ),
        out_shape=jax.ShapeDtypeStruct((PPAD, H), jnp.float32),
    )(off, xp, sw, Wg, Wu, Wd)

    out = _combine_sc(yw, slot.astype(jnp.int32))
    return out.reshape(b, s, h)


# speedup vs baseline: 2.1491x; 1.3782x over previous
import functools

import jax
import jax.numpy as jnp
from jax import lax
from jax.experimental import pallas as pl
from jax.experimental.pallas import tpu as pltpu
from jax.experimental.pallas import tpu_sc as plsc

B, S, H, I, E, TOP_K = 1, 2048, 768, 2048, 8, 2
T = B * S
P = T * TOP_K
TM = 256
TM2 = 128
NT = P // TM2 + E
PPAD = NT * TM2


def _router_body(x_ref, wr_ref, ei_ref, ew_ref):
    x = x_ref[...]
    logits = jnp.dot(x, wr_ref[...], preferred_element_type=jnp.float32)
    lmax = jnp.max(logits, axis=1, keepdims=True)
    z = jnp.exp(logits - lmax)
    iota = jax.lax.broadcasted_iota(jnp.int32, (TM, E), 1)
    e1 = jnp.argmax(z, axis=1)[:, None]
    m1 = jnp.max(z, axis=1, keepdims=True)
    z2 = jnp.where(iota == e1, -jnp.inf, z)
    e2 = jnp.argmax(z2, axis=1)[:, None]
    m2 = jnp.max(z2, axis=1, keepdims=True)
    denom = m1 + m2
    ei_ref[...] = jnp.concatenate(
        [e1, e2, jnp.zeros((TM, E - 2), jnp.int32)], axis=1)
    ew_ref[...] = jnp.concatenate(
        [m1 / denom, m2 / denom, jnp.zeros((TM, E - 2), jnp.float32)], axis=1)


def _expert_of(i, off_ref):
    acc = 0
    for e in range(E - 1):
        acc += (i * TM2 >= off_ref[e]).astype(jnp.int32)
    return acc


def _gemm_body(off_ref, xp_ref, wg_ref, wu_ref, wd_ref, y_ref):
    i = pl.program_id(0)

    @pl.when(i * TM2 < off_ref[E - 1])
    def _():
        xt = xp_ref[...]
        gate = jnp.dot(xt, wg_ref[...], preferred_element_type=jnp.float32)
        up = jnp.dot(xt, wu_ref[...], preferred_element_type=jnp.float32)
        inter = up * gate * jax.nn.sigmoid(gate)
        y_ref[...] = jnp.dot(inter, wd_ref[...],
                             preferred_element_type=jnp.float32)


def _bcast_lane(vec, lane):
    return lax.gather(
        vec, lane[:, None],
        dimension_numbers=lax.GatherDimensionNumbers(
            offset_dims=(), collapsed_slice_dims=(0,), start_index_map=(0,)),
        slice_sizes=(1,),
        mode=lax.GatherScatterMode.PROMISE_IN_BOUNDS)


def _bcast_lane_f32(vec, lane):
    return lax.gather(
        vec, lane[:, None],
        dimension_numbers=lax.GatherDimensionNumbers(
            offset_dims=(), collapsed_slice_dims=(0,), start_index_map=(0,)),
        slice_sizes=(1,),
        mode=lax.GatherScatterMode.PROMISE_IN_BOUNDS)


def _cumsum16(x, iota):
    r = x
    for sh in (1, 2, 4, 8):
        g = _bcast_lane(r, jnp.maximum(iota - sh, 0))
        r = r + jnp.where(iota >= sh, g, jnp.zeros((16,), jnp.int32))
    return r


W1 = 16
CPW = P // TM2 // W1


@functools.partial(
    pl.kernel,
    out_type=(
        jax.ShapeDtypeStruct((P // TM2, TM2), jnp.int32),
        jax.ShapeDtypeStruct((16,), jnp.int32),
        jax.ShapeDtypeStruct((PPAD, H), jnp.float32),
        jax.ShapeDtypeStruct((16, 16), jnp.int32),
    ),
    mesh=plsc.VectorSubcoreMesh(core_axis_name="c", subcore_axis_name="s"),
    scratch_types=[
        pltpu.VMEM((CPW, TM2), jnp.int32),
        pltpu.VMEM((CPW, TM2), jnp.int32),
        pltpu.VMEM((TM2, H), jnp.float32),
        pltpu.VMEM((16,), jnp.int32),
        pltpu.VMEM((16, 16), jnp.int32),
        pltpu.VMEM((16,), jnp.int32),
        pltpu.SemaphoreType.DMA,
        pltpu.SemaphoreType.DMA,
    ],
)
def _sort_sc(pe_hbm, x_hbm, slot_hbm, off_hbm, xp_hbm, cntx_hbm,
             pe_v, idx_v, rows_v, cnt_my, cnt_loc, off_v,
             sem, sem2):
    cid = lax.axis_index("c")
    wid = lax.axis_index("s")

    @pl.when(cid == 0)
    def _phase_count():
        iota = lax.iota(jnp.int32, 16)
        lane15 = jnp.full((16,), 15, jnp.int32)
        for sub in range(CPW):
            c = wid * CPW + sub
            pltpu.async_copy(
                pe_hbm.at[pl.ds(c * TM2, TM2)], pe_v.at[sub], sem2).wait()
        cntv = jnp.zeros((16,), jnp.int32)
        for sub in range(CPW):
            for i in range(TM2 // 16):
                v = pe_v[sub, pl.ds(i * 16, 16)]
                rank = jnp.zeros((16,), jnp.int32)
                newc = jnp.zeros((16,), jnp.int32)
                for e in range(E):
                    m = v == e
                    r = _cumsum16(jnp.where(m, 1, 0), iota)
                    rank = jnp.where(m, r, rank)
                    newc = jnp.where(iota == e, _bcast_lane(r, lane15), newc)
                idx_v[sub, pl.ds(i * 16, 16)] = rank - 1 + _bcast_lane(cntv, v)
                cntv = cntv + newc
        cnt_my[...] = cntv
        pltpu.sync_copy(cnt_my, cntx_hbm.at[wid])

    plsc.subcore_barrier()

    @pl.when(cid == 0)
    def _phase_rank_scatter():
        iota = lax.iota(jnp.int32, 16)
        pltpu.sync_copy(cntx_hbm, cnt_loc)
        total = jnp.zeros((16,), jnp.int32)
        for w2 in range(W1):
            total = total + cnt_loc[w2]
        padded = jnp.bitwise_and(total + (TM2 - 1), -TM2)
        incl = _cumsum16(padded, iota)
        base = incl - padded
        for w2 in range(W1):
            flag = jnp.where(jnp.int32(w2) < wid, 1, 0)
            base = base + cnt_loc[w2] * jnp.full((16,), flag, jnp.int32)

        @pl.when(wid == 0)
        def _():
            off_v[...] = incl
            pltpu.sync_copy(off_v, off_hbm)

        for sub in range(CPW):
            c = wid * CPW + sub
            t_base = lax.rem(c, jnp.int32(T // TM2)) * TM2
            rcopy = pltpu.async_copy(x_hbm.at[pl.ds(t_base, TM2)], rows_v, sem)
            for i in range(TM2 // 16):
                v = pe_v[sub, pl.ds(i * 16, 16)]
                lr = idx_v[sub, pl.ds(i * 16, 16)]
                idx_v[sub, pl.ds(i * 16, 16)] = lr + _bcast_lane(base, v)
            rcopy.wait()
            pltpu.async_copy(rows_v, xp_hbm.at[idx_v.at[sub]], sem).wait()
            pltpu.sync_copy(idx_v.at[sub], slot_hbm.at[c])


TPW = T // 32


@functools.partial(
    pl.kernel,
    out_type=jax.ShapeDtypeStruct((T, H), jnp.float32),
    mesh=plsc.VectorSubcoreMesh(core_axis_name="c", subcore_axis_name="s"),
    scratch_types=[
        pltpu.VMEM((TPW,), jnp.int32),
        pltpu.VMEM((TPW,), jnp.int32),
        pltpu.VMEM((TPW,), jnp.float32),
        pltpu.VMEM((TPW,), jnp.float32),
        pltpu.VMEM((TPW, H), jnp.float32),
        pltpu.VMEM((TPW, H), jnp.float32),
        pltpu.SemaphoreType.DMA,
        pltpu.SemaphoreType.DMA,
    ],
)
def _combine_sc(y_hbm, slot_hbm, pw_hbm, out_hbm,
                s0, s1, w0v, w1v, buf0, buf1, sem0, sem1):
    wid = lax.axis_index("s") * 2 + lax.axis_index("c")
    base = wid * TPW
    pltpu.sync_copy(slot_hbm.at[pl.ds(base, TPW)], s0)
    pltpu.sync_copy(slot_hbm.at[pl.ds(T + base, TPW)], s1)
    pltpu.sync_copy(pw_hbm.at[pl.ds(base, TPW)], w0v)
    pltpu.sync_copy(pw_hbm.at[pl.ds(T + base, TPW)], w1v)
    c0 = pltpu.async_copy(y_hbm.at[s0], buf0, sem0)
    c1 = pltpu.async_copy(y_hbm.at[s1], buf1, sem1)
    c0.wait()
    c1.wait()

    def body(j, carry):
        g = 16 * (j // 16)
        l = jnp.full((16,), lax.rem(j, 16), jnp.int32)
        w0s = _bcast_lane_f32(w0v[pl.ds(g, 16)], l)
        w1s = _bcast_lane_f32(w1v[pl.ds(g, 16)], l)
        for c in range(H // 16):
            a = buf0[j, pl.ds(c * 16, 16)]
            b = buf1[j, pl.ds(c * 16, 16)]
            buf0[j, pl.ds(c * 16, 16)] = w0s * a + w1s * b
        return carry

    lax.fori_loop(0, TPW, body, 0)
    pltpu.sync_copy(buf0, out_hbm.at[pl.ds(base, TPW)])


def kernel(hidden_states, Wr, Wg, Wu, Wd):
    b, s, h = hidden_states.shape
    x = hidden_states.reshape(T, H)

    ei, ew = pl.pallas_call(
        _router_body,
        grid=(T // TM,),
        in_specs=[
            pl.BlockSpec((TM, H), lambda i: (i, 0)),
            pl.BlockSpec((H, E), lambda i: (0, 0)),
        ],
        out_specs=[
            pl.BlockSpec((TM, E), lambda i: (i, 0)),
            pl.BlockSpec((TM, E), lambda i: (i, 0)),
        ],
        out_shape=[
            jax.ShapeDtypeStruct((T, E), jnp.int32),
            jax.ShapeDtypeStruct((T, E), jnp.float32),
        ],
    )(x, Wr)

    pe = jnp.concatenate([ei[:, 0], ei[:, 1]])
    pw = jnp.concatenate([ew[:, 0], ew[:, 1]])
    slot2d, off, xp, _cntx = _sort_sc(pe, x)
    slot = slot2d.reshape(P)

    yw = pl.pallas_call(
        _gemm_body,
        grid_spec=pltpu.PrefetchScalarGridSpec(
            num_scalar_prefetch=1,
            grid=(NT,),
            in_specs=[
                pl.BlockSpec((TM2, H), lambda i, off: (i, 0)),
                pl.BlockSpec((None, H, I), lambda i, off: (_expert_of(i, off), 0, 0)),
                pl.BlockSpec((None, H, I), lambda i, off: (_expert_of(i, off), 0, 0)),
                pl.BlockSpec((None, I, H), lambda i, off: (_expert_of(i, off), 0, 0)),
            ],
            out_specs=pl.BlockSpec((TM2, H), lambda i, off: (i, 0)),
        ),
        out_shape=jax.ShapeDtypeStruct((PPAD, H), jnp.float32),
    )(off, xp, Wg, Wu, Wd)

    out = _combine_sc(yw, slot, pw)
    return out.reshape(b, s, h)
